# Initial kernel scaffold; baseline (speedup 1.0000x reference)
import jax
import jax.numpy as jnp
from jax.experimental import pallas as pl


def kernel(x, W_flat, b, flip_vals, flip_idx):
    raise NotImplementedError("write your pallas kernel here")


# speedup vs baseline: 5.0813x; 5.0813x over previous
import functools

import jax
import jax.numpy as jnp
from jax import lax
from jax.experimental import pallas as pl
from jax.experimental.pallas import tpu as pltpu
from jax.experimental.pallas import tpu_sc as plsc

D_IN = 4096
D_OUT = 4096
NUMEL = D_OUT * D_IN

NC = 2
NS = 16
NW = NC * NS

LANE = 128
SLAB_ROWS = 16
SLAB = SLAB_ROWS * LANE


def _scatter_body(n_slabs, w_ref, idx_ref, val_ref, idx_v, val_v, sem):
    c = lax.axis_index("c")
    s = lax.axis_index("s")
    wid = s * NC + c
    row_base = wid * (n_slabs * SLAB_ROWS)

    def step(i, _):
        r0 = row_base + i * SLAB_ROWS
        pltpu.sync_copy(idx_ref.at[pl.ds(r0, SLAB_ROWS)], idx_v)
        pltpu.sync_copy(val_ref.at[pl.ds(r0, SLAB_ROWS)], val_v)
        copies = [
            pltpu.async_copy(val_v.at[j], w_ref.at[idx_v.at[j]], sem)
            for j in range(SLAB_ROWS)
        ]
        for cp in copies:
            cp.wait()
        return ()

    lax.fori_loop(0, n_slabs, step, ())


@functools.partial(jax.jit, static_argnames=("n_slabs",))
def _sc_scatter(w_ref, idx2d, val2d, *, n_slabs):
    mesh = plsc.VectorSubcoreMesh(
        core_axis_name="c", subcore_axis_name="s", num_cores=NC, num_subcores=NS
    )
    kern = pl.kernel(
        functools.partial(_scatter_body, n_slabs),
        out_type=(),
        mesh=mesh,
        scratch_types=[
            pltpu.VMEM((SLAB_ROWS, LANE), jnp.int32),
            pltpu.VMEM((SLAB_ROWS, LANE), jnp.float32),
            pltpu.SemaphoreType.DMA,
        ],
    )
    kern(w_ref, idx2d, val2d)


def _mm_body(x_ref, w_ref, b_ref, o_ref):
    xb = x_ref[...].astype(jnp.bfloat16)
    wb = w_ref[...].astype(jnp.bfloat16)
    acc = lax.dot_general(
        xb, wb, (((1,), (1,)), ((), ())), preferred_element_type=jnp.float32
    )
    o_ref[...] = acc + b_ref[...]


def _tc_matmul(x, w, b2d, interpret=False):
    n_blk = D_OUT // 256
    return pl.pallas_call(
        _mm_body,
        grid=(n_blk,),
        in_specs=[
            pl.BlockSpec((256, D_IN), lambda i: (0, 0)),
            pl.BlockSpec((256, D_IN), lambda i: (i, 0)),
            pl.BlockSpec((1, 256), lambda i: (0, i)),
        ],
        out_specs=pl.BlockSpec((256, 256), lambda i: (0, i)),
        out_shape=jax.ShapeDtypeStruct((256, D_OUT), jnp.float32),
        interpret=interpret,
    )(x, w, b2d)


def kernel(x, W_flat, b, flip_vals, flip_idx):
    n = flip_idx.shape[0]
    per_w = SLAB * ((n + NW * SLAB - 1) // (NW * SLAB))
    n_slabs = per_w // SLAB
    padded = per_w * NW
    pad = padded - n
    idx = flip_idx.astype(jnp.int32)
    idx_p = jnp.concatenate([idx, idx[:pad]]).reshape(-1, LANE)
    val_p = jnp.concatenate([flip_vals, flip_vals[:pad]]).reshape(-1, LANE)

    w_ref = jax.new_ref(W_flat)
    _sc_scatter(w_ref, idx_p, val_p, n_slabs=n_slabs)
    w_mod = w_ref[...].reshape(D_OUT, D_IN)
    return _tc_matmul(x, w_mod, b.reshape(1, D_OUT))


# speedup vs baseline: 5.2546x; 1.0341x over previous
import functools

import jax
import jax.numpy as jnp
from jax import lax
from jax.experimental import pallas as pl
from jax.experimental.pallas import tpu as pltpu
from jax.experimental.pallas import tpu_sc as plsc

D_IN = 4096
D_OUT = 4096
NUMEL = D_OUT * D_IN

NC = 2
NS = 16
NW = NC * NS

LANE = 128


def _scatter_body(per_w, w_ref, idx_ref, val_ref, idx_v, val_v, sem):
    c = lax.axis_index("c")
    s = lax.axis_index("s")
    wid = s * NC + c
    base = wid * per_w
    icp = pltpu.async_copy(idx_ref.at[pl.ds(base, per_w)], idx_v, sem)
    vcp = pltpu.async_copy(val_ref.at[pl.ds(base, per_w)], val_v, sem)
    icp.wait()
    vcp.wait()
    pltpu.sync_copy(val_v, w_ref.at[idx_v])


@functools.partial(jax.jit, static_argnames=("per_w",))
def _sc_scatter(w_ref, idx1d, val1d, *, per_w):
    mesh = plsc.VectorSubcoreMesh(
        core_axis_name="c", subcore_axis_name="s", num_cores=NC, num_subcores=NS
    )
    kern = pl.kernel(
        functools.partial(_scatter_body, per_w),
        out_type=(),
        mesh=mesh,
        scratch_types=[
            pltpu.VMEM((per_w,), jnp.int32),
            pltpu.VMEM((per_w,), jnp.float32),
            pltpu.SemaphoreType.DMA,
        ],
    )
    kern(w_ref, idx1d, val1d)


def _mm_body(x_ref, w_ref, b_ref, o_ref):
    xb = x_ref[...].astype(jnp.bfloat16)
    wb = w_ref[...].astype(jnp.bfloat16)
    acc = lax.dot_general(
        xb, wb, (((1,), (1,)), ((), ())), preferred_element_type=jnp.float32
    )
    o_ref[...] = acc + b_ref[...]


def _tc_matmul(x, w, b2d, interpret=False):
    n_blk = D_OUT // 256
    return pl.pallas_call(
        _mm_body,
        grid=(n_blk,),
        in_specs=[
            pl.BlockSpec((256, D_IN), lambda i: (0, 0)),
            pl.BlockSpec((256, D_IN), lambda i: (i, 0)),
            pl.BlockSpec((1, 256), lambda i: (0, i)),
        ],
        out_specs=pl.BlockSpec((256, 256), lambda i: (0, i)),
        out_shape=jax.ShapeDtypeStruct((256, D_OUT), jnp.float32),
        interpret=interpret,
    )(x, w, b2d)


def kernel(x, W_flat, b, flip_vals, flip_idx):
    n = flip_idx.shape[0]
    per_w = LANE * ((n + NW * LANE - 1) // (NW * LANE))
    padded = per_w * NW
    pad = padded - n
    idx = flip_idx.astype(jnp.int32)
    idx_p = jnp.concatenate([idx, idx[:pad]])
    val_p = jnp.concatenate([flip_vals, flip_vals[:pad]])

    w_ref = jax.new_ref(W_flat)
    _sc_scatter(w_ref, idx_p, val_p, per_w=per_w)
    w_mod = w_ref[...].reshape(D_OUT, D_IN)
    return _tc_matmul(x, w_mod, b.reshape(1, D_OUT))
